# baseline (speedup 1.0000x reference)
import functools

import jax
import jax.numpy as jnp
from jax import lax
from jax.experimental import pallas as pl
from jax.experimental.pallas import tpu as pltpu
from jax.experimental.pallas import tpu_sc as plsc

B = 512
D = 256
N = 4 * B
NUM_NEG = 128
ALPHA = 0.5
TILE = 256
GRID = N // TILE
MASK_FILL = -5.0

NW = 32
RPW = N // NW
NGRP = RPW // 16
NB = 512
BASE1 = -1.01
SCALE1 = NB / 2.02
SCALE2 = NB * SCALE1
KF = float(NUM_NEG)


def _dense_kernel(emb_ref, w1_ref, b1_ref, w2_ref, b2_ref, dw_ref, pos_ref,
                  sim_ref, m_ref, it_ref, ps_ref, reg_ref):
    i = pl.program_id(0)

    emb_full = emb_ref[...]
    nrm = jnp.sqrt(jnp.sum(emb_full * emb_full, axis=1, keepdims=True))
    emb_n = emb_full / jnp.maximum(nrm, 1e-12)

    row0 = i * TILE
    tile_raw = emb_ref[pl.ds(row0, TILE), :]
    tile_nrm = jnp.sqrt(jnp.sum(tile_raw * tile_raw, axis=1, keepdims=True))
    tile_n = tile_raw / jnp.maximum(tile_nrm, 1e-12)

    sim = lax.dot_general(tile_n, emb_n, (((1,), (1,)), ((), ())),
                          preferred_element_type=jnp.float32)

    h = jnp.maximum(jnp.dot(tile_raw, w1_ref[...],
                            preferred_element_type=jnp.float32)
                    + b1_ref[...], 0.0)
    tlin = jnp.dot(h, w2_ref[...], preferred_element_type=jnp.float32) \
        + b2_ref[...]
    temps = 0.01 + 0.99 * jax.nn.sigmoid(tlin)
    it_ref[...] = 1.0 / temps

    local = row0 % B + lax.broadcasted_iota(jnp.int32, (TILE, 1), 0)
    pr = pos_ref[pl.ds(row0, TILE), :]
    pos_local = pr + (pr >= local).astype(jnp.int32)
    pos_idx = (row0 // B) * B + pos_local

    col = lax.broadcasted_iota(jnp.int32, (TILE, N), 1)
    pos_sim = jnp.sum(jnp.where(col == pos_idx, sim, 0.0), axis=1,
                      keepdims=True)
    ps_ref[...] = pos_sim

    dom = row0 // B
    masked = jnp.where((col // B) == dom, MASK_FILL, sim)
    sim_ref[...] = masked

    row_max = jnp.max(masked, axis=1, keepdims=True)
    m_ref[...] = jnp.maximum(pos_sim, row_max)

    @pl.when(i == 0)
    def _():
        cent = jnp.mean(emb_full.reshape(4, B, D), axis=1)
        reg = jnp.zeros((1, 1), jnp.float32)
        for a in range(4):
            for b in range(a + 1, 4):
                dvec = cent[a] - cent[b]
                reg = reg + dw_ref[a, b] * jnp.sqrt(jnp.sum(dvec * dvec))
        reg_ref[...] = reg / 6.0


def _mine_body(sim_hbm, m_hbm, it_hbm, out_hbm,
               buf_v, m_v, it_v, h1_v, h2_v, out_v):
    wid = lax.axis_index("s") * 2 + lax.axis_index("c")
    row0 = wid * RPW
    pltpu.sync_copy(m_hbm.at[pl.ds(row0, RPW)], m_v)
    pltpu.sync_copy(it_hbm.at[pl.ds(row0, RPW)], it_v)

    lanes = lax.broadcasted_iota(jnp.int32, (16,), 0)
    ones = jnp.ones((16,), jnp.float32)
    zf = jnp.zeros((16,), jnp.float32)
    zi = jnp.zeros((16,), jnp.int32)

    for g in range(NGRP):
        r0 = row0 + g * 16
        pltpu.sync_copy(sim_hbm.at[pl.ds(r0 * N, 16 * N)], buf_v)
        m16 = m_v[pl.ds(g * 16, 16)]
        it16 = it_v[pl.ds(g * 16, 16)]

        def zero_hists(b, _):
            h1_v[pl.ds(b * 16, 16)] = zf
            h2_v[pl.ds(b * 16, 16)] = zf
            return 0
        lax.fori_loop(0, NB, zero_hists, 0)

        def pass_a(j, _):
            v = plsc.load_gather(buf_v, [lanes * N + j])
            b1 = jnp.clip(((v - BASE1) * SCALE1).astype(jnp.int32), 0, NB - 1)
            plsc.addupdate_scatter(h1_v, [b1 * 16 + lanes], ones)
            return 0
        lax.fori_loop(0, N, pass_a, 0)

        def scan1(t, carry):
            cum, bsel, ca = carry
            bb = NB - 1 - t
            hh = h1_v[pl.ds(bb * 16, 16)]
            new = cum + hh
            hit = (new >= KF) & (cum < KF)
            return new, jnp.where(hit, bb, bsel), jnp.where(hit, cum, ca)
        _, b1sel, ca1 = lax.fori_loop(0, NB, scan1, (zf, zi, zf))
        lo1 = BASE1 + b1sel.astype(jnp.float32) * (1.0 / SCALE1)

        def pass_b(j, _):
            v = plsc.load_gather(buf_v, [lanes * N + j])
            b1 = jnp.clip(((v - BASE1) * SCALE1).astype(jnp.int32), 0, NB - 1)
            b2 = jnp.clip(((v - lo1) * SCALE2).astype(jnp.int32), 0, NB - 1)
            plsc.addupdate_scatter(h2_v, [b2 * 16 + lanes], ones,
                                   mask=b1 == b1sel)
            return 0
        lax.fori_loop(0, N, pass_b, 0)

        th = KF - ca1

        def scan2(t, carry):
            cum, bsel, ca = carry
            bb = NB - 1 - t
            hh = h2_v[pl.ds(bb * 16, 16)]
            new = cum + hh
            hit = (new >= th) & (cum < th)
            return new, jnp.where(hit, bb, bsel), jnp.where(hit, cum, ca)
        _, b2sel, ca2 = lax.fori_loop(0, NB, scan2, (zf, zi, zf))

        t_rep = lo1 + (b2sel.astype(jnp.float32) + 0.5) * (1.0 / SCALE2)
        count_above = ca1 + ca2

        def pass_c(j, acc):
            v = plsc.load_gather(buf_v, [lanes * N + j])
            b1 = jnp.clip(((v - BASE1) * SCALE1).astype(jnp.int32), 0, NB - 1)
            b2 = jnp.clip(((v - lo1) * SCALE2).astype(jnp.int32), 0, NB - 1)
            in_top = (b1 > b1sel) | ((b1 == b1sel) & (b2 > b2sel))
            e = jnp.exp((v - m16) * it16)
            return acc + jnp.where(in_top, e, 0.0)
        acc = lax.fori_loop(0, N, pass_c, zf)

        out_v[pl.ds(g * 16, 16)] = acc + (KF - count_above) * jnp.exp(
            (t_rep - m16) * it16)

    pltpu.sync_copy(out_v, out_hbm.at[pl.ds(row0, RPW)])


_mine = functools.partial(
    pl.kernel,
    mesh=plsc.VectorSubcoreMesh(core_axis_name="c", subcore_axis_name="s"),
    out_type=jax.ShapeDtypeStruct((N,), jnp.float32),
    compiler_params=pltpu.CompilerParams(needs_layout_passes=False),
    scratch_types=[
        pltpu.VMEM((16 * N,), jnp.float32),
        pltpu.VMEM((RPW,), jnp.float32),
        pltpu.VMEM((RPW,), jnp.float32),
        pltpu.VMEM((NB * 16,), jnp.float32),
        pltpu.VMEM((NB * 16,), jnp.float32),
        pltpu.VMEM((RPW,), jnp.float32),
    ],
)(_mine_body)


def _final_kernel(st_ref, ps_ref, m_ref, it_ref, hw_ref, loss_ref):
    m = m_ref[...]
    ps = ps_ref[...]
    it = it_ref[...]
    total = jnp.exp((ps - m) * it) + st_ref[...]
    losses = ((m - ps) * it + jnp.log(total)) * hw_ref[...]
    loss_ref[...] = jnp.sum(losses).reshape(1, 1)


@jax.jit
def _run(all_emb, w1, b1, w2, b2, dw, hw, pos_rand):
    whole = lambda x: pl.BlockSpec(x.shape, lambda i: (0,) * x.ndim)
    args = (all_emb, w1, b1.reshape(1, 64), w2, b2.reshape(1, 1), dw,
            pos_rand.reshape(N, 1))
    col1 = jax.ShapeDtypeStruct((N, 1), jnp.float32)
    sim, m, it, ps, reg = pl.pallas_call(
        _dense_kernel,
        grid=(GRID,),
        in_specs=[whole(a) for a in args],
        out_specs=[
            pl.BlockSpec((TILE, N), lambda i: (i, 0)),
            pl.BlockSpec((TILE, 1), lambda i: (i, 0)),
            pl.BlockSpec((TILE, 1), lambda i: (i, 0)),
            pl.BlockSpec((TILE, 1), lambda i: (i, 0)),
            pl.BlockSpec((1, 1), lambda i: (0, 0)),
        ],
        out_shape=[
            jax.ShapeDtypeStruct((N, N), jnp.float32),
            col1, col1, col1,
            jax.ShapeDtypeStruct((1, 1), jnp.float32),
        ],
    )(*args)

    sum_top = _mine(sim.reshape(-1), m.reshape(-1), it.reshape(-1))

    fargs = (sum_top.reshape(N, 1), ps, m, it, hw.reshape(N, 1))
    loss_sum = pl.pallas_call(
        _final_kernel,
        in_specs=[pl.BlockSpec(a.shape, None) for a in fargs],
        out_specs=pl.BlockSpec((1, 1), None),
        out_shape=jax.ShapeDtypeStruct((1, 1), jnp.float32),
    )(*fargs)
    return loss_sum[0, 0] / N + ALPHA * reg[0, 0]


def kernel(emb_vision, emb_nlp, emb_security, emb_medical, hard_sample_weights,
           W1, b1, W2, b2, domain_weights, domain_ids, pos_rand):
    all_emb = jnp.concatenate([emb_vision, emb_nlp, emb_security, emb_medical],
                              axis=0)
    return _run(all_emb, W1, b1, W2, b2, domain_weights, hard_sample_weights,
                pos_rand)


# speedup vs baseline: 2.5461x; 2.5461x over previous
import functools

import jax
import jax.numpy as jnp
from jax import lax
from jax.experimental import pallas as pl
from jax.experimental.pallas import tpu as pltpu
from jax.experimental.pallas import tpu_sc as plsc

B = 512
D = 256
N = 4 * B
NUM_NEG = 128
ALPHA = 0.5
TILE = 256
GRID = N // TILE
MASK_FILL = -5.0

NW = 32
RPW = N // NW
NGRP = RPW // 16
NB = 256
BASE1 = -1.01
SCALE1 = NB / 2.02
SCALE2 = NB * SCALE1
KF = float(NUM_NEG)
STEP = 8


def _dense_kernel(emb_ref, w1_ref, b1_ref, w2_ref, b2_ref, dw_ref, pos_ref,
                  sim_ref, m_ref, it_ref, ps_ref, reg_ref):
    i = pl.program_id(0)

    emb_full = emb_ref[...]
    nrm = jnp.sqrt(jnp.sum(emb_full * emb_full, axis=1, keepdims=True))
    emb_n = emb_full / jnp.maximum(nrm, 1e-12)

    row0 = i * TILE
    tile_raw = emb_ref[pl.ds(row0, TILE), :]
    tile_nrm = jnp.sqrt(jnp.sum(tile_raw * tile_raw, axis=1, keepdims=True))
    tile_n = tile_raw / jnp.maximum(tile_nrm, 1e-12)

    sim = lax.dot_general(tile_n, emb_n, (((1,), (1,)), ((), ())),
                          preferred_element_type=jnp.float32)

    h = jnp.maximum(jnp.dot(tile_raw, w1_ref[...],
                            preferred_element_type=jnp.float32)
                    + b1_ref[...], 0.0)
    tlin = jnp.dot(h, w2_ref[...], preferred_element_type=jnp.float32) \
        + b2_ref[...]
    temps = 0.01 + 0.99 * jax.nn.sigmoid(tlin)
    it_ref[...] = 1.0 / temps

    local = row0 % B + lax.broadcasted_iota(jnp.int32, (TILE, 1), 0)
    pr = pos_ref[pl.ds(row0, TILE), :]
    pos_local = pr + (pr >= local).astype(jnp.int32)
    pos_idx = (row0 // B) * B + pos_local

    col = lax.broadcasted_iota(jnp.int32, (TILE, N), 1)
    pos_sim = jnp.sum(jnp.where(col == pos_idx, sim, 0.0), axis=1,
                      keepdims=True)
    ps_ref[...] = pos_sim

    dom = row0 // B
    masked = jnp.where((col // B) == dom, MASK_FILL, sim)
    sim_ref[...] = masked

    row_max = jnp.max(masked, axis=1, keepdims=True)
    m_ref[...] = jnp.maximum(pos_sim, row_max)

    @pl.when(i == 0)
    def _():
        cent = jnp.mean(emb_full.reshape(4, B, D), axis=1)
        reg = jnp.zeros((1, 1), jnp.float32)
        for a in range(4):
            for b in range(a + 1, 4):
                dvec = cent[a] - cent[b]
                reg = reg + dw_ref[a, b] * jnp.sqrt(jnp.sum(dvec * dvec))
        reg_ref[...] = reg / 6.0


def _mine_body(sim_hbm, m_hbm, it_hbm, out_hbm,
               buf_v, m_v, it_v, h1_v, hc_v, he_v, out_v):
    wid = lax.axis_index("s") * 2 + lax.axis_index("c")
    row0 = wid * RPW
    pltpu.sync_copy(m_hbm.at[pl.ds(row0, RPW)], m_v)
    pltpu.sync_copy(it_hbm.at[pl.ds(row0, RPW)], it_v)

    lanes = lax.broadcasted_iota(jnp.int32, (16,), 0)
    lanes_n = lanes * N
    ones = jnp.ones((16,), jnp.float32)
    zf = jnp.zeros((16,), jnp.float32)
    zi = jnp.zeros((16,), jnp.int32)

    for g in range(NGRP):
        r0 = row0 + g * 16
        pltpu.sync_copy(sim_hbm.at[pl.ds(r0 * N, 16 * N)], buf_v)
        m16 = m_v[pl.ds(g * 16, 16)]
        it16 = it_v[pl.ds(g * 16, 16)]

        @plsc.parallel_loop(0, NB, 1, unroll=8)
        def _zero(b):
            h1_v[pl.ds(b * 16, 16)] = zf
            hc_v[pl.ds(b * 16, 16)] = zf
            he_v[pl.ds(b * 16, 16)] = zf

        @plsc.parallel_loop(0, N, STEP, unroll=2)
        def _pass1(j):
            for t in range(STEP):
                v = plsc.load_gather(buf_v, [lanes_n + (j + t)])
                b1 = jnp.clip(((v - BASE1) * SCALE1).astype(jnp.int32),
                              0, NB - 1)
                plsc.addupdate_scatter(h1_v, [(b1 << 4) + lanes], ones)

        def scan1(t, carry):
            cum, bsel, ca = carry
            bb = NB - 1 - t
            hh = h1_v[pl.ds(bb * 16, 16)]
            new = cum + hh
            hit = (new >= KF) & (cum < KF)
            return new, jnp.where(hit, bb, bsel), jnp.where(hit, cum, ca)
        _, b1sel, ca1 = plsc.parallel_loop(0, NB, 1, unroll=4,
                                           carry=(zf, zi, zf))(scan1)
        lo1 = BASE1 + b1sel.astype(jnp.float32) * (1.0 / SCALE1)

        def pass2(j, accs):
            new = []
            for t in range(STEP):
                v = plsc.load_gather(buf_v, [lanes_n + (j + t)])
                b1 = jnp.clip(((v - BASE1) * SCALE1).astype(jnp.int32),
                              0, NB - 1)
                e = jnp.exp((v - m16) * it16)
                new.append(accs[t] + jnp.where(b1 > b1sel, e, 0.0))
                b2 = jnp.clip(((v - lo1) * SCALE2).astype(jnp.int32),
                              0, NB - 1)
                idx2 = (b2 << 4) + lanes
                inb = b1 == b1sel
                plsc.addupdate_scatter(hc_v, [idx2], ones, mask=inb)
                plsc.addupdate_scatter(he_v, [idx2], e, mask=inb)
            return tuple(new)
        accs = plsc.parallel_loop(0, N, STEP, unroll=2,
                                  carry=(zf,) * STEP)(pass2)
        acc_above = accs[0] + accs[1] + accs[2] + accs[3] \
            + accs[4] + accs[5] + accs[6] + accs[7]

        th = KF - ca1

        def scan2(t, carry):
            cum, bsel, ca, ecum, esel = carry
            bb = NB - 1 - t
            hh = hc_v[pl.ds(bb * 16, 16)]
            ee = he_v[pl.ds(bb * 16, 16)]
            new = cum + hh
            hit = (new >= th) & (cum < th)
            return (new, jnp.where(hit, bb, bsel), jnp.where(hit, cum, ca),
                    ecum + ee, jnp.where(hit, ecum, esel))
        _, b2sel, ca2, _, esel = plsc.parallel_loop(
            0, NB, 1, unroll=4, carry=(zf, zi, zf, zf, zf))(scan2)

        t_rep = lo1 + (b2sel.astype(jnp.float32) + 0.5) * (1.0 / SCALE2)
        out_v[pl.ds(g * 16, 16)] = acc_above + esel \
            + (KF - ca1 - ca2) * jnp.exp((t_rep - m16) * it16)

    pltpu.sync_copy(out_v, out_hbm.at[pl.ds(row0, RPW)])


_mine = functools.partial(
    pl.kernel,
    mesh=plsc.VectorSubcoreMesh(core_axis_name="c", subcore_axis_name="s"),
    out_type=jax.ShapeDtypeStruct((N,), jnp.float32),
    compiler_params=pltpu.CompilerParams(needs_layout_passes=False),
    scratch_types=[
        pltpu.VMEM((16 * N,), jnp.float32),
        pltpu.VMEM((RPW,), jnp.float32),
        pltpu.VMEM((RPW,), jnp.float32),
        pltpu.VMEM((NB * 16,), jnp.float32),
        pltpu.VMEM((NB * 16,), jnp.float32),
        pltpu.VMEM((NB * 16,), jnp.float32),
        pltpu.VMEM((RPW,), jnp.float32),
    ],
)(_mine_body)


def _final_kernel(st_ref, ps_ref, m_ref, it_ref, hw_ref, loss_ref):
    m = m_ref[...]
    ps = ps_ref[...]
    it = it_ref[...]
    total = jnp.exp((ps - m) * it) + st_ref[...]
    losses = ((m - ps) * it + jnp.log(total)) * hw_ref[...]
    loss_ref[...] = jnp.sum(losses).reshape(1, 1)


@jax.jit
def _run(all_emb, w1, b1, w2, b2, dw, hw, pos_rand):
    whole = lambda x: pl.BlockSpec(x.shape, lambda i: (0,) * x.ndim)
    args = (all_emb, w1, b1.reshape(1, 64), w2, b2.reshape(1, 1), dw,
            pos_rand.reshape(N, 1))
    col1 = jax.ShapeDtypeStruct((N, 1), jnp.float32)
    sim, m, it, ps, reg = pl.pallas_call(
        _dense_kernel,
        grid=(GRID,),
        in_specs=[whole(a) for a in args],
        out_specs=[
            pl.BlockSpec((TILE, N), lambda i: (i, 0)),
            pl.BlockSpec((TILE, 1), lambda i: (i, 0)),
            pl.BlockSpec((TILE, 1), lambda i: (i, 0)),
            pl.BlockSpec((TILE, 1), lambda i: (i, 0)),
            pl.BlockSpec((1, 1), lambda i: (0, 0)),
        ],
        out_shape=[
            jax.ShapeDtypeStruct((N, N), jnp.float32),
            col1, col1, col1,
            jax.ShapeDtypeStruct((1, 1), jnp.float32),
        ],
    )(*args)

    sum_top = _mine(sim.reshape(-1), m.reshape(-1), it.reshape(-1))

    fargs = (sum_top.reshape(N, 1), ps, m, it, hw.reshape(N, 1))
    loss_sum = pl.pallas_call(
        _final_kernel,
        in_specs=[pl.BlockSpec(a.shape, None) for a in fargs],
        out_specs=pl.BlockSpec((1, 1), None),
        out_shape=jax.ShapeDtypeStruct((1, 1), jnp.float32),
    )(*fargs)
    return loss_sum[0, 0] / N + ALPHA * reg[0, 0]


def kernel(emb_vision, emb_nlp, emb_security, emb_medical, hard_sample_weights,
           W1, b1, W2, b2, domain_weights, domain_ids, pos_rand):
    all_emb = jnp.concatenate([emb_vision, emb_nlp, emb_security, emb_medical],
                              axis=0)
    return _run(all_emb, W1, b1, W2, b2, domain_weights, hard_sample_weights,
                pos_rand)
